# Initial kernel scaffold; baseline (speedup 1.0000x reference)
import jax
import jax.numpy as jnp
from jax.experimental import pallas as pl


def kernel(x, edge_index, Wq1, bq1, Wk1, bk1, Wv1, bv1, Ws1, bs1, Wb1, Wq2, bq2, Wk2, bk2, Wv2, bv2, Ws2, bs2, Wb2, ln_g, ln_b):
    raise NotImplementedError("write your pallas kernel here")


# speedup vs baseline: 26.2913x; 26.2913x over previous
import dataclasses
import functools

import jax
import jax.numpy as jnp
import numpy as np
from jax import lax
from jax.experimental import pallas as pl
from jax.experimental.pallas import tpu as pltpu
from jax.experimental.pallas import tpu_sc as plsc

N = 10000
D = 128
H = 4
C = 24
HC = H * C
E = 320000
AW = 128
QW = 128
KVW = 256

NC = 2
NS = 16
L = 16
NW = NC * NS
EPW = E // NW
B = 80
NCHUNK = EPW // B
DR = 624

NB = 1000
GRID = N // NB

_PREC = jax.lax.Precision.HIGHEST


def _edge_body(q_hbm, kv_hbm, dst_hbm, src_hbm, out_hbm,
               dstbuf, srcbuf, qbuf, kvbuf, wbuf, sbuf, sbufi, flagbuf,
               idxg, acc, sem0, sem1):
    core = lax.axis_index("c")
    sub = lax.axis_index("s")
    wid = core * NS + sub

    @pl.loop(0, B)
    def _zw(i):
        for j in range(AW // L):
            wbuf[i, pl.ds(j * L, L)] = jnp.zeros((L,), jnp.float32)

    base_r = sub * DR
    for t in range(DR // B):
        pltpu.sync_copy(wbuf, acc.at[pl.ds(base_r + t * B, B)])
    pltpu.sync_copy(wbuf.at[pl.ds(0, DR - (DR // B) * B)],
                    acc.at[pl.ds(base_r + (DR // B) * B, DR - (DR // B) * B)])

    @pl.when(sub == NS - 1)
    def _ztail():
        pltpu.sync_copy(wbuf.at[pl.ds(0, N - NS * DR)],
                        acc.at[pl.ds(NS * DR, N - NS * DR)])

    plsc.subcore_barrier()

    ji = lax.iota(jnp.int32, L)
    idx8 = lax.bitwise_and(ji + 8, 15)
    idx4 = lax.bitwise_and(ji + 4, 15)

    @pl.loop(0, NCHUNK)
    def _chunk(ci):
        base = wid * EPW + ci * B
        pltpu.sync_copy(dst_hbm.at[pl.ds(base, B)], dstbuf)
        pltpu.sync_copy(src_hbm.at[pl.ds(base, B)], srcbuf)
        cq = pltpu.async_copy(q_hbm.at[dstbuf], qbuf, sem0)
        ckv = pltpu.async_copy(kv_hbm.at[srcbuf], kvbuf, sem1)
        cq.wait()
        ckv.wait()

        @pl.loop(0, B)
        def _edge(e):
            s = qbuf[e, pl.ds(0, L)] * kvbuf[e, pl.ds(0, L)]
            for j in range(1, HC // L):
                s = s + qbuf[e, pl.ds(j * L, L)] * kvbuf[e, pl.ds(j * L, L)]
            sbuf[...] = s
            s = s + plsc.load_gather(sbuf, [idx8])
            sbuf[...] = s
            s = s + plsc.load_gather(sbuf, [idx4])
            ex = jnp.exp(s)
            for j in range(HC // L):
                wbuf[e, pl.ds(j * L, L)] = kvbuf[e, pl.ds(HC + j * L, L)] * ex
            wbuf[e, pl.ds(HC, L)] = ex
            wbuf[e, pl.ds(HC + L, L)] = jnp.zeros((L,), jnp.float32)

        ji2 = lax.iota(jnp.int32, L)
        idxm1 = lax.bitwise_and(ji2 + 15, 15)
        one = jnp.full((L,), 1, jnp.int32)
        zero = jnp.full((L,), 0, jnp.int32)

        def _dup_of_earlier(keys):
            dk, pv = plsc.sort_key_val(keys, ji2)
            sbufi[...] = dk
            prev = plsc.load_gather(sbufi, [idxm1])
            adj = jnp.where((dk == prev) & (ji2 > 0), one, zero)
            plsc.store_scatter(flagbuf, [pv], adj)
            return flagbuf[...] == 1

        for g in range(B // L):
            d = dstbuf[pl.ds(g * L, L)]
            dup = _dup_of_earlier(d)
            idxg[...] = jnp.where(dup, N + ji2, d)
            pltpu.sync_copy(wbuf.at[pl.ds(g * L, L)], acc.at[idxg], add=True)
            def _patch_pass(rem_i32):
                rem = rem_i32 == 1
                keys = jnp.where(rem, d, N + L + ji2)
                dup2 = _dup_of_earlier(keys)
                resolved = rem & jnp.logical_not(dup2)
                idxg[...] = jnp.where(resolved, d, N + ji2)
                pltpu.sync_copy(wbuf.at[pl.ds(g * L, L)],
                                acc.at[idxg], add=True)
                return jnp.where(rem & dup2, one, zero)

            lax.while_loop(lambda r: jnp.any(r == 1), _patch_pass,
                           jnp.where(dup, one, zero))

    plsc.subcore_barrier()
    for t in range(DR // B):
        r0 = base_r + t * B
        pltpu.sync_copy(acc.at[pl.ds(r0, B)], out_hbm.at[core, pl.ds(r0, B)])
    r0 = base_r + (DR // B) * B
    pltpu.sync_copy(acc.at[pl.ds(r0, DR - (DR // B) * B)],
                    out_hbm.at[core, pl.ds(r0, DR - (DR // B) * B)])

    @pl.when(sub == NS - 1)
    def _dtail():
        pltpu.sync_copy(acc.at[pl.ds(NS * DR, N - NS * DR)],
                        out_hbm.at[core, pl.ds(NS * DR, N - NS * DR)])


@functools.lru_cache(maxsize=None)
def _build_edge_pass():
    cp = pltpu.CompilerParams()
    if "needs_layout_passes" in pltpu.CompilerParams.__dataclass_fields__:
        cp = dataclasses.replace(cp, needs_layout_passes=False)
    return pl.kernel(
        _edge_body,
        out_type=jax.ShapeDtypeStruct((NC, N, AW), jnp.float32),
        compiler_params=cp,
        mesh=plsc.VectorSubcoreMesh(core_axis_name="c", subcore_axis_name="s",
                                    num_cores=NC, num_subcores=NS),
        scratch_types=[
            pltpu.VMEM((B,), jnp.int32),
            pltpu.VMEM((B,), jnp.int32),
            pltpu.VMEM((B, QW), jnp.float32),
            pltpu.VMEM((B, KVW), jnp.float32),
            pltpu.VMEM((B, AW), jnp.float32),
            pltpu.VMEM((L,), jnp.float32),
            pltpu.VMEM((L,), jnp.int32),
            pltpu.VMEM((L,), jnp.int32),
            pltpu.VMEM((L,), jnp.int32),
            pltpu.VMEM_SHARED((N + L, AW), jnp.float32),
            pltpu.SemaphoreType.DMA,
            pltpu.SemaphoreType.DMA,
        ],
    )


def _edge_pass(q, kv, dst, src):
    return _build_edge_pass()(q, kv, dst, src)


def _store_padded(q_ref, kv_ref, y):
    zq = jnp.zeros((y.shape[0], QW - HC), jnp.float32)
    zkv = jnp.zeros((y.shape[0], KVW - 2 * HC), jnp.float32)
    q_ref[...] = jnp.concatenate([y[:, :HC], zq], axis=1)
    kv_ref[...] = jnp.concatenate([y[:, HC:], zkv], axis=1)


def _proj1_body(x_ref, w_ref, b_ref, q_ref, kv_ref):
    y = jnp.dot(x_ref[...], w_ref[...], preferred_element_type=jnp.float32,
                precision=_PREC) + b_ref[...]
    _store_padded(q_ref, kv_ref, y)


def _combine(acc_ref, xin_ref, texp_ref, ws_ref, bs_ref, wba_ref, wbb_ref):
    a = acc_ref[0] + acc_ref[1]
    den = jnp.dot(a[:, HC:HC + 16], texp_ref[...],
                  preferred_element_type=jnp.float32, precision=_PREC) + 1e-16
    out = a[:, :HC] / den
    xr = jnp.dot(xin_ref[...], ws_ref[...], preferred_element_type=jnp.float32,
                 precision=_PREC) + bs_ref[...]
    g = (jnp.sum(out * wba_ref[...], axis=1, keepdims=True)
         + jnp.sum(xr * wbb_ref[...], axis=1, keepdims=True))
    beta = jax.nn.sigmoid(g)
    return beta * xr + (1.0 - beta) * out


def _mid_body(acc_ref, x_ref, texp_ref, ws_ref, bs_ref, wba_ref, wbb_ref,
              w2_ref, b2_ref, h_ref, q2_ref, kv2_ref):
    hp = _combine(acc_ref, x_ref, texp_ref, ws_ref, bs_ref, wba_ref, wbb_ref)
    h = 0.5 * hp * (1.0 + lax.erf(hp * np.float32(1.0 / np.sqrt(2.0))))
    h_ref[...] = h
    y2 = jnp.dot(h, w2_ref[...], preferred_element_type=jnp.float32,
                 precision=_PREC) + b2_ref[...]
    _store_padded(q2_ref, kv2_ref, y2)


def _fin_body(acc_ref, h_ref, texp_ref, ws_ref, bs_ref, wba_ref, wbb_ref,
              g_ref, bln_ref, o_ref):
    hp = _combine(acc_ref, h_ref, texp_ref, ws_ref, bs_ref, wba_ref, wbb_ref)
    mu = jnp.mean(hp, axis=1, keepdims=True)
    var = jnp.mean((hp - mu) ** 2, axis=1, keepdims=True)
    o_ref[...] = (hp - mu) / jnp.sqrt(var + 1e-5) * g_ref[...] + bln_ref[...]


def _full(shape):
    return pl.BlockSpec(shape, lambda i: (0,) * len(shape))


@functools.lru_cache(maxsize=None)
def _build_tc_kernels(interpret=False):
    proj1 = pl.pallas_call(
        _proj1_body,
        grid=(GRID,),
        in_specs=[
            pl.BlockSpec((NB, D), lambda i: (i, 0)),
            _full((D, 3 * HC)),
            _full((1, 3 * HC)),
        ],
        out_specs=[
            pl.BlockSpec((NB, QW), lambda i: (i, 0)),
            pl.BlockSpec((NB, KVW), lambda i: (i, 0)),
        ],
        out_shape=[
            jax.ShapeDtypeStruct((N, QW), jnp.float32),
            jax.ShapeDtypeStruct((N, KVW), jnp.float32),
        ],
        interpret=interpret,
    )

    mid = pl.pallas_call(
        _mid_body,
        grid=(GRID,),
        in_specs=[
            pl.BlockSpec((NC, NB, AW), lambda i: (0, i, 0)),
            pl.BlockSpec((NB, D), lambda i: (i, 0)),
            _full((16, HC)),
            _full((D, HC)),
            _full((1, HC)),
            _full((1, HC)),
            _full((1, HC)),
            _full((HC, 3 * HC)),
            _full((1, 3 * HC)),
        ],
        out_specs=[
            pl.BlockSpec((NB, HC), lambda i: (i, 0)),
            pl.BlockSpec((NB, QW), lambda i: (i, 0)),
            pl.BlockSpec((NB, KVW), lambda i: (i, 0)),
        ],
        out_shape=[
            jax.ShapeDtypeStruct((N, HC), jnp.float32),
            jax.ShapeDtypeStruct((N, QW), jnp.float32),
            jax.ShapeDtypeStruct((N, KVW), jnp.float32),
        ],
        interpret=interpret,
    )

    fin = pl.pallas_call(
        _fin_body,
        grid=(GRID,),
        in_specs=[
            pl.BlockSpec((NC, NB, AW), lambda i: (0, i, 0)),
            pl.BlockSpec((NB, HC), lambda i: (i, 0)),
            _full((16, HC)),
            _full((HC, HC)),
            _full((1, HC)),
            _full((1, HC)),
            _full((1, HC)),
            _full((1, HC)),
            _full((1, HC)),
        ],
        out_specs=pl.BlockSpec((NB, HC), lambda i: (i, 0)),
        out_shape=jax.ShapeDtypeStruct((N, HC), jnp.float32),
        interpret=interpret,
    )
    return proj1, mid, fin


def _pcol(w):
    return w.reshape(-1, H, C).transpose(0, 2, 1).reshape(w.shape[0], HC)


def _pvec(b):
    return b.reshape(H, C).T.reshape(HC)


def _prow(w):
    return w.reshape(H, C, -1).transpose(1, 0, 2).reshape(HC, w.shape[1])


def kernel(x, edge_index, Wq1, bq1, Wk1, bk1, Wv1, bv1, Ws1, bs1, Wb1,
           Wq2, bq2, Wk2, bk2, Wv2, bv2, Ws2, bs2, Wb2, ln_g, ln_b):
    src = edge_index[0]
    dst = edge_index[1]
    sc = np.float32(1.0 / np.sqrt(C))

    w1 = jnp.concatenate([_pcol(Wq1) * sc, _pcol(Wk1), _pcol(Wv1)], axis=1)
    b1 = jnp.concatenate([_pvec(bq1) * sc, _pvec(bk1), _pvec(bv1)])[None, :]
    ws1, bs1p = _pcol(Ws1), _pvec(bs1)[None, :]
    wba1 = _pvec(Wb1[:HC, 0] + Wb1[2 * HC:, 0])[None, :]
    wbb1 = _pvec(Wb1[HC:2 * HC, 0] - Wb1[2 * HC:, 0])[None, :]

    w2 = jnp.concatenate([_pcol(_prow(Wq2)) * sc, _pcol(_prow(Wk2)),
                          _pcol(_prow(Wv2))], axis=1)
    b2 = jnp.concatenate([_pvec(bq2) * sc, _pvec(bk2), _pvec(bv2)])[None, :]
    ws2, bs2p = _pcol(_prow(Ws2)), _pvec(bs2)[None, :]
    wba2 = _pvec(Wb2[:HC, 0] + Wb2[2 * HC:, 0])[None, :]
    wbb2 = _pvec(Wb2[HC:2 * HC, 0] - Wb2[2 * HC:, 0])[None, :]

    lg, lb = _pvec(ln_g)[None, :], _pvec(ln_b)[None, :]

    li = np.arange(16)[:, None]
    ci = np.arange(HC)[None, :]
    texp = jnp.asarray((li % 4 == ci % 4) * 0.25, dtype=jnp.float32)

    _proj1, _mid, _fin = _build_tc_kernels()

    q1, kv1 = _proj1(x, w1, b1)
    acc1 = _edge_pass(q1, kv1, dst, src)
    h, q2, kv2 = _mid(acc1, x, texp, ws1, bs1p, wba1, wbb1, w2, b2)
    acc2 = _edge_pass(q2, kv2, dst, src)
    yp = _fin(acc2, h, texp, ws2, bs2p, wba2, wbb2, lg, lb)
    return yp.reshape(N, C, H).transpose(0, 2, 1).reshape(N, HC)


# speedup vs baseline: 27.6279x; 1.0508x over previous
import dataclasses
import functools

import jax
import jax.numpy as jnp
import numpy as np
from jax import lax
from jax.experimental import pallas as pl
from jax.experimental.pallas import tpu as pltpu
from jax.experimental.pallas import tpu_sc as plsc

N = 10000
D = 128
H = 4
C = 24
HC = H * C
E = 320000
AW = 128
QW = 128
KVW = 256

NC = 2
NS = 16
L = 16
NW = NC * NS
EPW = E // NW
B = 80
NCHUNK = EPW // B
DR = 624

NB = 1000
GRID = N // NB

_PREC = jax.lax.Precision.HIGHEST


def _edge_body(q_hbm, kv_hbm, dst_hbm, src_hbm, out_hbm,
               dstbuf, srcbuf, qbuf, kvbuf, wbuf, sbuf, sbufi, flagbuf,
               idxg, idxp, acc, sem0, sem1, sems):
    core = lax.axis_index("c")
    sub = lax.axis_index("s")
    wid = core * NS + sub

    @pl.loop(0, B)
    def _zw(i):
        for j in range(AW // L):
            wbuf[i, pl.ds(j * L, L)] = jnp.zeros((L,), jnp.float32)

    base_r = sub * DR
    for t in range(DR // B):
        pltpu.sync_copy(wbuf, acc.at[pl.ds(base_r + t * B, B)])
    pltpu.sync_copy(wbuf.at[pl.ds(0, DR - (DR // B) * B)],
                    acc.at[pl.ds(base_r + (DR // B) * B, DR - (DR // B) * B)])

    @pl.when(sub == NS - 1)
    def _ztail():
        pltpu.sync_copy(wbuf.at[pl.ds(0, N - NS * DR)],
                        acc.at[pl.ds(NS * DR, N - NS * DR)])

    plsc.subcore_barrier()

    ji = lax.iota(jnp.int32, L)
    idx8 = lax.bitwise_and(ji + 8, 15)
    idx4 = lax.bitwise_and(ji + 4, 15)

    @pl.loop(0, NCHUNK)
    def _chunk(ci):
        base = wid * EPW + ci * B
        pltpu.sync_copy(dst_hbm.at[pl.ds(base, B)], dstbuf)
        pltpu.sync_copy(src_hbm.at[pl.ds(base, B)], srcbuf)
        cq = pltpu.async_copy(q_hbm.at[dstbuf], qbuf, sem0)
        ckv = pltpu.async_copy(kv_hbm.at[srcbuf], kvbuf, sem1)
        cq.wait()
        ckv.wait()

        @pl.loop(0, B)
        def _edge(e):
            s = qbuf[e, pl.ds(0, L)] * kvbuf[e, pl.ds(0, L)]
            for j in range(1, HC // L):
                s = s + qbuf[e, pl.ds(j * L, L)] * kvbuf[e, pl.ds(j * L, L)]
            sbuf[...] = s
            s = s + plsc.load_gather(sbuf, [idx8])
            sbuf[...] = s
            s = s + plsc.load_gather(sbuf, [idx4])
            ex = jnp.exp(s)
            for j in range(HC // L):
                wbuf[e, pl.ds(j * L, L)] = kvbuf[e, pl.ds(HC + j * L, L)] * ex
            wbuf[e, pl.ds(HC, L)] = ex
            wbuf[e, pl.ds(HC + L, L)] = jnp.zeros((L,), jnp.float32)

        ji2 = lax.iota(jnp.int32, L)
        idxm1 = lax.bitwise_and(ji2 + 15, 15)
        one = jnp.full((L,), 1, jnp.int32)
        zero = jnp.full((L,), 0, jnp.int32)

        def _dup_of_earlier(keys):
            dk, pv = plsc.sort_key_val(keys, ji2)
            sbufi[...] = dk
            prev = plsc.load_gather(sbufi, [idxm1])
            adj = jnp.where((dk == prev) & (ji2 > 0), one, zero)
            plsc.store_scatter(flagbuf, [pv], adj)
            return flagbuf[...] == 1

        cps = []
        for g in range(B // L):
            d = dstbuf[pl.ds(g * L, L)]
            dup = _dup_of_earlier(d)
            idxg[g, pl.ds(0, L)] = jnp.where(dup, N + ji2, d)
            cps.append(pltpu.async_copy(wbuf.at[pl.ds(g * L, L)],
                                        acc.at[idxg.at[g]], sems, add=True))
            def _patch_pass(rem_i32):
                rem = rem_i32 == 1
                keys = jnp.where(rem, d, N + L + ji2)
                dup2 = _dup_of_earlier(keys)
                resolved = rem & jnp.logical_not(dup2)
                idxp[...] = jnp.where(resolved, d, N + ji2)
                pltpu.sync_copy(wbuf.at[pl.ds(g * L, L)],
                                acc.at[idxp], add=True)
                return jnp.where(rem & dup2, one, zero)

            lax.while_loop(lambda r: jnp.any(r == 1), _patch_pass,
                           jnp.where(dup, one, zero))
        for cp in cps:
            cp.wait()

    plsc.subcore_barrier()
    for t in range(DR // B):
        r0 = base_r + t * B
        pltpu.sync_copy(acc.at[pl.ds(r0, B)], out_hbm.at[core, pl.ds(r0, B)])
    r0 = base_r + (DR // B) * B
    pltpu.sync_copy(acc.at[pl.ds(r0, DR - (DR // B) * B)],
                    out_hbm.at[core, pl.ds(r0, DR - (DR // B) * B)])

    @pl.when(sub == NS - 1)
    def _dtail():
        pltpu.sync_copy(acc.at[pl.ds(NS * DR, N - NS * DR)],
                        out_hbm.at[core, pl.ds(NS * DR, N - NS * DR)])


@functools.lru_cache(maxsize=None)
def _build_edge_pass():
    cp = pltpu.CompilerParams()
    if "needs_layout_passes" in pltpu.CompilerParams.__dataclass_fields__:
        cp = dataclasses.replace(cp, needs_layout_passes=False)
    return pl.kernel(
        _edge_body,
        out_type=jax.ShapeDtypeStruct((NC, N, AW), jnp.float32),
        compiler_params=cp,
        mesh=plsc.VectorSubcoreMesh(core_axis_name="c", subcore_axis_name="s",
                                    num_cores=NC, num_subcores=NS),
        scratch_types=[
            pltpu.VMEM((B,), jnp.int32),
            pltpu.VMEM((B,), jnp.int32),
            pltpu.VMEM((B, QW), jnp.float32),
            pltpu.VMEM((B, KVW), jnp.float32),
            pltpu.VMEM((B, AW), jnp.float32),
            pltpu.VMEM((L,), jnp.float32),
            pltpu.VMEM((L,), jnp.int32),
            pltpu.VMEM((L,), jnp.int32),
            pltpu.VMEM((B // L, L), jnp.int32),
            pltpu.VMEM((L,), jnp.int32),
            pltpu.VMEM_SHARED((N + L, AW), jnp.float32),
            pltpu.SemaphoreType.DMA,
            pltpu.SemaphoreType.DMA,
            pltpu.SemaphoreType.DMA,
        ],
    )


def _edge_pass(q, kv, dst, src):
    return _build_edge_pass()(q, kv, dst, src)


def _store_padded(q_ref, kv_ref, y):
    zq = jnp.zeros((y.shape[0], QW - HC), jnp.float32)
    zkv = jnp.zeros((y.shape[0], KVW - 2 * HC), jnp.float32)
    q_ref[...] = jnp.concatenate([y[:, :HC], zq], axis=1)
    kv_ref[...] = jnp.concatenate([y[:, HC:], zkv], axis=1)


def _proj1_body(x_ref, w_ref, b_ref, q_ref, kv_ref):
    y = jnp.dot(x_ref[...], w_ref[...], preferred_element_type=jnp.float32,
                precision=_PREC) + b_ref[...]
    _store_padded(q_ref, kv_ref, y)


def _combine(acc_ref, xin_ref, texp_ref, ws_ref, bs_ref, wba_ref, wbb_ref):
    a = acc_ref[0] + acc_ref[1]
    den = jnp.dot(a[:, HC:HC + 16], texp_ref[...],
                  preferred_element_type=jnp.float32, precision=_PREC) + 1e-16
    out = a[:, :HC] / den
    xr = jnp.dot(xin_ref[...], ws_ref[...], preferred_element_type=jnp.float32,
                 precision=_PREC) + bs_ref[...]
    g = (jnp.sum(out * wba_ref[...], axis=1, keepdims=True)
         + jnp.sum(xr * wbb_ref[...], axis=1, keepdims=True))
    beta = jax.nn.sigmoid(g)
    return beta * xr + (1.0 - beta) * out


def _mid_body(acc_ref, x_ref, texp_ref, ws_ref, bs_ref, wba_ref, wbb_ref,
              w2_ref, b2_ref, h_ref, q2_ref, kv2_ref):
    hp = _combine(acc_ref, x_ref, texp_ref, ws_ref, bs_ref, wba_ref, wbb_ref)
    h = 0.5 * hp * (1.0 + lax.erf(hp * np.float32(1.0 / np.sqrt(2.0))))
    h_ref[...] = h
    y2 = jnp.dot(h, w2_ref[...], preferred_element_type=jnp.float32,
                 precision=_PREC) + b2_ref[...]
    _store_padded(q2_ref, kv2_ref, y2)


def _fin_body(acc_ref, h_ref, texp_ref, ws_ref, bs_ref, wba_ref, wbb_ref,
              g_ref, bln_ref, o_ref):
    hp = _combine(acc_ref, h_ref, texp_ref, ws_ref, bs_ref, wba_ref, wbb_ref)
    mu = jnp.mean(hp, axis=1, keepdims=True)
    var = jnp.mean((hp - mu) ** 2, axis=1, keepdims=True)
    o_ref[...] = (hp - mu) / jnp.sqrt(var + 1e-5) * g_ref[...] + bln_ref[...]


def _full(shape):
    return pl.BlockSpec(shape, lambda i: (0,) * len(shape))


@functools.lru_cache(maxsize=None)
def _build_tc_kernels(interpret=False):
    proj1 = pl.pallas_call(
        _proj1_body,
        grid=(GRID,),
        in_specs=[
            pl.BlockSpec((NB, D), lambda i: (i, 0)),
            _full((D, 3 * HC)),
            _full((1, 3 * HC)),
        ],
        out_specs=[
            pl.BlockSpec((NB, QW), lambda i: (i, 0)),
            pl.BlockSpec((NB, KVW), lambda i: (i, 0)),
        ],
        out_shape=[
            jax.ShapeDtypeStruct((N, QW), jnp.float32),
            jax.ShapeDtypeStruct((N, KVW), jnp.float32),
        ],
        interpret=interpret,
    )

    mid = pl.pallas_call(
        _mid_body,
        grid=(GRID,),
        in_specs=[
            pl.BlockSpec((NC, NB, AW), lambda i: (0, i, 0)),
            pl.BlockSpec((NB, D), lambda i: (i, 0)),
            _full((16, HC)),
            _full((D, HC)),
            _full((1, HC)),
            _full((1, HC)),
            _full((1, HC)),
            _full((HC, 3 * HC)),
            _full((1, 3 * HC)),
        ],
        out_specs=[
            pl.BlockSpec((NB, HC), lambda i: (i, 0)),
            pl.BlockSpec((NB, QW), lambda i: (i, 0)),
            pl.BlockSpec((NB, KVW), lambda i: (i, 0)),
        ],
        out_shape=[
            jax.ShapeDtypeStruct((N, HC), jnp.float32),
            jax.ShapeDtypeStruct((N, QW), jnp.float32),
            jax.ShapeDtypeStruct((N, KVW), jnp.float32),
        ],
        interpret=interpret,
    )

    fin = pl.pallas_call(
        _fin_body,
        grid=(GRID,),
        in_specs=[
            pl.BlockSpec((NC, NB, AW), lambda i: (0, i, 0)),
            pl.BlockSpec((NB, HC), lambda i: (i, 0)),
            _full((16, HC)),
            _full((HC, HC)),
            _full((1, HC)),
            _full((1, HC)),
            _full((1, HC)),
            _full((1, HC)),
            _full((1, HC)),
        ],
        out_specs=pl.BlockSpec((NB, HC), lambda i: (i, 0)),
        out_shape=jax.ShapeDtypeStruct((N, HC), jnp.float32),
        interpret=interpret,
    )
    return proj1, mid, fin


def _pcol(w):
    return w.reshape(-1, H, C).transpose(0, 2, 1).reshape(w.shape[0], HC)


def _pvec(b):
    return b.reshape(H, C).T.reshape(HC)


def _prow(w):
    return w.reshape(H, C, -1).transpose(1, 0, 2).reshape(HC, w.shape[1])


def kernel(x, edge_index, Wq1, bq1, Wk1, bk1, Wv1, bv1, Ws1, bs1, Wb1,
           Wq2, bq2, Wk2, bk2, Wv2, bv2, Ws2, bs2, Wb2, ln_g, ln_b):
    src = edge_index[0]
    dst = edge_index[1]
    sc = np.float32(1.0 / np.sqrt(C))

    w1 = jnp.concatenate([_pcol(Wq1) * sc, _pcol(Wk1), _pcol(Wv1)], axis=1)
    b1 = jnp.concatenate([_pvec(bq1) * sc, _pvec(bk1), _pvec(bv1)])[None, :]
    ws1, bs1p = _pcol(Ws1), _pvec(bs1)[None, :]
    wba1 = _pvec(Wb1[:HC, 0] + Wb1[2 * HC:, 0])[None, :]
    wbb1 = _pvec(Wb1[HC:2 * HC, 0] - Wb1[2 * HC:, 0])[None, :]

    w2 = jnp.concatenate([_pcol(_prow(Wq2)) * sc, _pcol(_prow(Wk2)),
                          _pcol(_prow(Wv2))], axis=1)
    b2 = jnp.concatenate([_pvec(bq2) * sc, _pvec(bk2), _pvec(bv2)])[None, :]
    ws2, bs2p = _pcol(_prow(Ws2)), _pvec(bs2)[None, :]
    wba2 = _pvec(Wb2[:HC, 0] + Wb2[2 * HC:, 0])[None, :]
    wbb2 = _pvec(Wb2[HC:2 * HC, 0] - Wb2[2 * HC:, 0])[None, :]

    lg, lb = _pvec(ln_g)[None, :], _pvec(ln_b)[None, :]

    li = np.arange(16)[:, None]
    ci = np.arange(HC)[None, :]
    texp = jnp.asarray((li % 4 == ci % 4) * 0.25, dtype=jnp.float32)

    _proj1, _mid, _fin = _build_tc_kernels()

    q1, kv1 = _proj1(x, w1, b1)
    acc1 = _edge_pass(q1, kv1, dst, src)
    h, q2, kv2 = _mid(acc1, x, texp, ws1, bs1p, wba1, wbb1, w2, b2)
    acc2 = _edge_pass(q2, kv2, dst, src)
    yp = _fin(acc2, h, texp, ws2, bs2p, wba2, wbb2, lg, lb)
    return yp.reshape(N, C, H).transpose(0, 2, 1).reshape(N, HC)
